# baseline (speedup 1.0000x reference)
import jax
import jax.numpy as jnp
from jax.experimental import pallas as pl

N = 5000
IN = 512
H1 = 256
H2 = 64

_BM1 = 512
_BM2 = 256
_BM3 = 512
_BT = 512


def _xw_body(x_ref, w0_ref, w1_ref, p0_ref, p1_ref):
    xb = x_ref[...].astype(jnp.bfloat16)
    w0 = w0_ref[...].astype(jnp.bfloat16)
    w1 = w1_ref[...].astype(jnp.bfloat16)
    p0_ref[...] = jnp.dot(xb, w0, preferred_element_type=jnp.float32).astype(jnp.bfloat16)
    p1_ref[...] = jnp.dot(xb, w1, preferred_element_type=jnp.float32).astype(jnp.bfloat16)


def _layer1_body(a0_ref, a1_ref, p0_ref, p1_ref, w2_ref, mw2_ref):
    a0 = a0_ref[0].astype(jnp.bfloat16)
    a1 = a1_ref[0].astype(jnp.bfloat16)
    h0 = jnp.maximum(jnp.dot(a0, p0_ref[...], preferred_element_type=jnp.float32), 0.0)
    h1 = jnp.maximum(jnp.dot(a1, p1_ref[...], preferred_element_type=jnp.float32), 0.0)
    mean = ((h0 + h1) * 0.5).astype(jnp.bfloat16)
    w2 = w2_ref[...].astype(jnp.bfloat16)
    mw2_ref[...] = jnp.dot(mean, w2, preferred_element_type=jnp.float32).astype(jnp.bfloat16)


def _layer2_body(a0_ref, mw2_ref, emb_ref, embb_ref, embt_ref):
    a0 = a0_ref[0].astype(jnp.bfloat16)
    e = jnp.maximum(jnp.dot(a0, mw2_ref[...], preferred_element_type=jnp.float32), 0.0)
    emb_ref[...] = e
    eb = e.astype(jnp.bfloat16)
    embb_ref[...] = eb
    embt_ref[...] = eb.T


def _recon_body(ei_ref, etj_ref, out_ref):
    out_ref[...] = jnp.dot(ei_ref[...], etj_ref[...], preferred_element_type=jnp.float32)


def kernel(x, adjs, W0, W1, W2):
    p0, p1 = pl.pallas_call(
        _xw_body,
        grid=(pl.cdiv(N, _BM1),),
        in_specs=[
            pl.BlockSpec((_BM1, IN), lambda i: (i, 0)),
            pl.BlockSpec((IN, H1), lambda i: (0, 0)),
            pl.BlockSpec((IN, H1), lambda i: (0, 0)),
        ],
        out_specs=[
            pl.BlockSpec((_BM1, H1), lambda i: (i, 0)),
            pl.BlockSpec((_BM1, H1), lambda i: (i, 0)),
        ],
        out_shape=[
            jax.ShapeDtypeStruct((N, H1), jnp.bfloat16),
            jax.ShapeDtypeStruct((N, H1), jnp.bfloat16),
        ],
    )(x, W0, W1)

    mw2 = pl.pallas_call(
        _layer1_body,
        grid=(pl.cdiv(N, _BM2),),
        in_specs=[
            pl.BlockSpec((1, _BM2, N), lambda i: (0, i, 0)),
            pl.BlockSpec((1, _BM2, N), lambda i: (1, i, 0)),
            pl.BlockSpec((N, H1), lambda i: (0, 0)),
            pl.BlockSpec((N, H1), lambda i: (0, 0)),
            pl.BlockSpec((H1, H2), lambda i: (0, 0)),
        ],
        out_specs=pl.BlockSpec((_BM2, H2), lambda i: (i, 0)),
        out_shape=jax.ShapeDtypeStruct((N, H2), jnp.bfloat16),
    )(adjs, adjs, p0, p1, W2)

    emb, embb, embt = pl.pallas_call(
        _layer2_body,
        grid=(pl.cdiv(N, _BM3),),
        in_specs=[
            pl.BlockSpec((1, _BM3, N), lambda i: (0, i, 0)),
            pl.BlockSpec((N, H2), lambda i: (0, 0)),
        ],
        out_specs=[
            pl.BlockSpec((_BM3, H2), lambda i: (i, 0)),
            pl.BlockSpec((_BM3, H2), lambda i: (i, 0)),
            pl.BlockSpec((H2, _BM3), lambda i: (0, i)),
        ],
        out_shape=[
            jax.ShapeDtypeStruct((N, H2), jnp.float32),
            jax.ShapeDtypeStruct((N, H2), jnp.bfloat16),
            jax.ShapeDtypeStruct((H2, N), jnp.bfloat16),
        ],
    )(adjs, mw2)

    recon = pl.pallas_call(
        _recon_body,
        grid=(pl.cdiv(N, _BT), pl.cdiv(N, _BT)),
        in_specs=[
            pl.BlockSpec((_BT, H2), lambda i, j: (i, 0)),
            pl.BlockSpec((H2, _BT), lambda i, j: (0, j)),
        ],
        out_specs=pl.BlockSpec((_BT, _BT), lambda i, j: (i, j)),
        out_shape=jax.ShapeDtypeStruct((N, N), jnp.float32),
    )(embb, embt)

    return emb, recon, recon


# speedup vs baseline: 1.2038x; 1.2038x over previous
import jax
import jax.numpy as jnp
from jax.experimental import pallas as pl

N = 5000
IN = 512
H1 = 256
H2 = 64

_BM1 = 512
_BM2 = 256
_BM3 = 512
_BT = 512


def _xw_body(x_ref, w0_ref, w1_ref, p0_ref, p1_ref):
    xb = x_ref[...].astype(jnp.bfloat16)
    w0 = w0_ref[...].astype(jnp.bfloat16)
    w1 = w1_ref[...].astype(jnp.bfloat16)
    p0_ref[...] = jnp.dot(xb, w0, preferred_element_type=jnp.float32).astype(jnp.bfloat16)
    p1_ref[...] = jnp.dot(xb, w1, preferred_element_type=jnp.float32).astype(jnp.bfloat16)


def _layer1_body(a0_ref, a1_ref, p0_ref, p1_ref, w2_ref, mw2_ref):
    a0 = a0_ref[0].astype(jnp.bfloat16)
    a1 = a1_ref[0].astype(jnp.bfloat16)
    h0 = jnp.maximum(jnp.dot(a0, p0_ref[...], preferred_element_type=jnp.float32), 0.0)
    h1 = jnp.maximum(jnp.dot(a1, p1_ref[...], preferred_element_type=jnp.float32), 0.0)
    mean = ((h0 + h1) * 0.5).astype(jnp.bfloat16)
    w2 = w2_ref[...].astype(jnp.bfloat16)
    mw2_ref[...] = jnp.dot(mean, w2, preferred_element_type=jnp.float32).astype(jnp.bfloat16)


def _layer2_body(a0_ref, mw2_ref, emb_ref, embb_ref, embt_ref):
    a0 = a0_ref[0].astype(jnp.bfloat16)
    e = jnp.maximum(jnp.dot(a0, mw2_ref[...], preferred_element_type=jnp.float32), 0.0)
    emb_ref[...] = e
    eb = e.astype(jnp.bfloat16)
    embb_ref[...] = eb
    embt_ref[...] = eb.T


def _recon_body(ei_ref, etj_ref, out0_ref, out1_ref):
    r = jnp.dot(ei_ref[...], etj_ref[...], preferred_element_type=jnp.float32)
    out0_ref[...] = r
    out1_ref[...] = r


def kernel(x, adjs, W0, W1, W2):
    p0, p1 = pl.pallas_call(
        _xw_body,
        grid=(pl.cdiv(N, _BM1),),
        in_specs=[
            pl.BlockSpec((_BM1, IN), lambda i: (i, 0)),
            pl.BlockSpec((IN, H1), lambda i: (0, 0)),
            pl.BlockSpec((IN, H1), lambda i: (0, 0)),
        ],
        out_specs=[
            pl.BlockSpec((_BM1, H1), lambda i: (i, 0)),
            pl.BlockSpec((_BM1, H1), lambda i: (i, 0)),
        ],
        out_shape=[
            jax.ShapeDtypeStruct((N, H1), jnp.bfloat16),
            jax.ShapeDtypeStruct((N, H1), jnp.bfloat16),
        ],
    )(x, W0, W1)

    mw2 = pl.pallas_call(
        _layer1_body,
        grid=(pl.cdiv(N, _BM2),),
        in_specs=[
            pl.BlockSpec((1, _BM2, N), lambda i: (0, i, 0)),
            pl.BlockSpec((1, _BM2, N), lambda i: (1, i, 0)),
            pl.BlockSpec((N, H1), lambda i: (0, 0)),
            pl.BlockSpec((N, H1), lambda i: (0, 0)),
            pl.BlockSpec((H1, H2), lambda i: (0, 0)),
        ],
        out_specs=pl.BlockSpec((_BM2, H2), lambda i: (i, 0)),
        out_shape=jax.ShapeDtypeStruct((N, H2), jnp.bfloat16),
    )(adjs, adjs, p0, p1, W2)

    emb, embb, embt = pl.pallas_call(
        _layer2_body,
        grid=(pl.cdiv(N, _BM3),),
        in_specs=[
            pl.BlockSpec((1, _BM3, N), lambda i: (0, i, 0)),
            pl.BlockSpec((N, H2), lambda i: (0, 0)),
        ],
        out_specs=[
            pl.BlockSpec((_BM3, H2), lambda i: (i, 0)),
            pl.BlockSpec((_BM3, H2), lambda i: (i, 0)),
            pl.BlockSpec((H2, _BM3), lambda i: (0, i)),
        ],
        out_shape=[
            jax.ShapeDtypeStruct((N, H2), jnp.float32),
            jax.ShapeDtypeStruct((N, H2), jnp.bfloat16),
            jax.ShapeDtypeStruct((H2, N), jnp.bfloat16),
        ],
    )(adjs, mw2)

    recon0, recon1 = pl.pallas_call(
        _recon_body,
        grid=(pl.cdiv(N, _BT), pl.cdiv(N, _BT)),
        in_specs=[
            pl.BlockSpec((_BT, H2), lambda i, j: (i, 0)),
            pl.BlockSpec((H2, _BT), lambda i, j: (0, j)),
        ],
        out_specs=[
            pl.BlockSpec((_BT, _BT), lambda i, j: (i, j)),
            pl.BlockSpec((_BT, _BT), lambda i, j: (i, j)),
        ],
        out_shape=[
            jax.ShapeDtypeStruct((N, N), jnp.float32),
            jax.ShapeDtypeStruct((N, N), jnp.float32),
        ],
    )(embb, embt)

    return emb, recon0, recon1


# speedup vs baseline: 1.3508x; 1.1222x over previous
import jax
import jax.numpy as jnp
from jax.experimental import pallas as pl

N = 5000
IN = 512
H1 = 256
H2 = 64

_BM1 = 512
_BM2 = 512
_BM3 = 1024
_BTI = 512
_BTJ = 1024


def _xw_body(x_ref, w0_ref, w1_ref, p0_ref, p1_ref):
    xb = x_ref[...].astype(jnp.bfloat16)
    w0 = w0_ref[...].astype(jnp.bfloat16)
    w1 = w1_ref[...].astype(jnp.bfloat16)
    p0_ref[...] = jnp.dot(xb, w0, preferred_element_type=jnp.float32).astype(jnp.bfloat16)
    p1_ref[...] = jnp.dot(xb, w1, preferred_element_type=jnp.float32).astype(jnp.bfloat16)


def _layer1_body(a0_ref, a1_ref, p0_ref, p1_ref, w2_ref, mw2_ref):
    a0 = a0_ref[0].astype(jnp.bfloat16)
    a1 = a1_ref[0].astype(jnp.bfloat16)
    h0 = jnp.maximum(jnp.dot(a0, p0_ref[...], preferred_element_type=jnp.float32), 0.0)
    h1 = jnp.maximum(jnp.dot(a1, p1_ref[...], preferred_element_type=jnp.float32), 0.0)
    mean = ((h0 + h1) * 0.5).astype(jnp.bfloat16)
    w2 = w2_ref[...].astype(jnp.bfloat16)
    mw2_ref[...] = jnp.dot(mean, w2, preferred_element_type=jnp.float32).astype(jnp.bfloat16)


def _layer2_body(a0_ref, mw2_ref, emb_ref, embb_ref, embt_ref):
    a0 = a0_ref[0].astype(jnp.bfloat16)
    e = jnp.maximum(jnp.dot(a0, mw2_ref[...], preferred_element_type=jnp.float32), 0.0)
    emb_ref[...] = e
    eb = e.astype(jnp.bfloat16)
    embb_ref[...] = eb
    embt_ref[...] = eb.T


def _recon_body(ei_ref, etj_ref, out0_ref, out1_ref):
    r = jnp.dot(ei_ref[...], etj_ref[...], preferred_element_type=jnp.float32)
    out0_ref[...] = r
    out1_ref[...] = r


def kernel(x, adjs, W0, W1, W2):
    p0, p1 = pl.pallas_call(
        _xw_body,
        grid=(pl.cdiv(N, _BM1),),
        in_specs=[
            pl.BlockSpec((_BM1, IN), lambda i: (i, 0)),
            pl.BlockSpec((IN, H1), lambda i: (0, 0)),
            pl.BlockSpec((IN, H1), lambda i: (0, 0)),
        ],
        out_specs=[
            pl.BlockSpec((_BM1, H1), lambda i: (i, 0)),
            pl.BlockSpec((_BM1, H1), lambda i: (i, 0)),
        ],
        out_shape=[
            jax.ShapeDtypeStruct((N, H1), jnp.bfloat16),
            jax.ShapeDtypeStruct((N, H1), jnp.bfloat16),
        ],
    )(x, W0, W1)

    mw2 = pl.pallas_call(
        _layer1_body,
        grid=(pl.cdiv(N, _BM2),),
        in_specs=[
            pl.BlockSpec((1, _BM2, N), lambda i: (0, i, 0)),
            pl.BlockSpec((1, _BM2, N), lambda i: (1, i, 0)),
            pl.BlockSpec((N, H1), lambda i: (0, 0)),
            pl.BlockSpec((N, H1), lambda i: (0, 0)),
            pl.BlockSpec((H1, H2), lambda i: (0, 0)),
        ],
        out_specs=pl.BlockSpec((_BM2, H2), lambda i: (i, 0)),
        out_shape=jax.ShapeDtypeStruct((N, H2), jnp.bfloat16),
    )(adjs, adjs, p0, p1, W2)

    emb, embb, embt = pl.pallas_call(
        _layer2_body,
        grid=(pl.cdiv(N, _BM3),),
        in_specs=[
            pl.BlockSpec((1, _BM3, N), lambda i: (0, i, 0)),
            pl.BlockSpec((N, H2), lambda i: (0, 0)),
        ],
        out_specs=[
            pl.BlockSpec((_BM3, H2), lambda i: (i, 0)),
            pl.BlockSpec((_BM3, H2), lambda i: (i, 0)),
            pl.BlockSpec((H2, _BM3), lambda i: (0, i)),
        ],
        out_shape=[
            jax.ShapeDtypeStruct((N, H2), jnp.float32),
            jax.ShapeDtypeStruct((N, H2), jnp.bfloat16),
            jax.ShapeDtypeStruct((H2, N), jnp.bfloat16),
        ],
    )(adjs, mw2)

    recon0, recon1 = pl.pallas_call(
        _recon_body,
        grid=(pl.cdiv(N, _BTI), pl.cdiv(N, _BTJ)),
        in_specs=[
            pl.BlockSpec((_BTI, H2), lambda i, j: (i, 0)),
            pl.BlockSpec((H2, _BTJ), lambda i, j: (0, j)),
        ],
        out_specs=[
            pl.BlockSpec((_BTI, _BTJ), lambda i, j: (i, j)),
            pl.BlockSpec((_BTI, _BTJ), lambda i, j: (i, j)),
        ],
        out_shape=[
            jax.ShapeDtypeStruct((N, N), jnp.float32),
            jax.ShapeDtypeStruct((N, N), jnp.float32),
        ],
    )(embb, embt)

    return emb, recon0, recon1


# speedup vs baseline: 1.4105x; 1.0442x over previous
import jax
import jax.numpy as jnp
from jax.experimental import pallas as pl

N = 5000
IN = 512
H1 = 256
H2 = 64

_BM1 = 512
_BM2 = 512
_BM3 = 1024
_BTI = 1024
_BTJ = 1280


def _xw_body(x_ref, w0_ref, w1_ref, p0_ref, p1_ref):
    xb = x_ref[...].astype(jnp.bfloat16)
    w0 = w0_ref[...].astype(jnp.bfloat16)
    w1 = w1_ref[...].astype(jnp.bfloat16)
    p0_ref[...] = jnp.dot(xb, w0, preferred_element_type=jnp.float32).astype(jnp.bfloat16)
    p1_ref[...] = jnp.dot(xb, w1, preferred_element_type=jnp.float32).astype(jnp.bfloat16)


def _layer1_body(a0_ref, a1_ref, p0_ref, p1_ref, w2_ref, mw2_ref):
    a0 = a0_ref[0].astype(jnp.bfloat16)
    a1 = a1_ref[0].astype(jnp.bfloat16)
    h0 = jnp.maximum(jnp.dot(a0, p0_ref[...], preferred_element_type=jnp.float32), 0.0)
    h1 = jnp.maximum(jnp.dot(a1, p1_ref[...], preferred_element_type=jnp.float32), 0.0)
    mean = ((h0 + h1) * 0.5).astype(jnp.bfloat16)
    w2 = w2_ref[...].astype(jnp.bfloat16)
    mw2_ref[...] = jnp.dot(mean, w2, preferred_element_type=jnp.float32).astype(jnp.bfloat16)


def _layer2_body(a0_ref, mw2_ref, emb_ref, embb_ref, embt_ref):
    a0 = a0_ref[0].astype(jnp.bfloat16)
    e = jnp.maximum(jnp.dot(a0, mw2_ref[...], preferred_element_type=jnp.float32), 0.0)
    emb_ref[...] = e
    eb = e.astype(jnp.bfloat16)
    embb_ref[...] = eb
    embt_ref[...] = eb.T


def _recon_body(ei_ref, etj_ref, out0_ref, out1_ref):
    r = jnp.dot(ei_ref[...], etj_ref[...], preferred_element_type=jnp.float32)
    out0_ref[...] = r
    out1_ref[...] = r


def kernel(x, adjs, W0, W1, W2):
    p0, p1 = pl.pallas_call(
        _xw_body,
        grid=(pl.cdiv(N, _BM1),),
        in_specs=[
            pl.BlockSpec((_BM1, IN), lambda i: (i, 0)),
            pl.BlockSpec((IN, H1), lambda i: (0, 0)),
            pl.BlockSpec((IN, H1), lambda i: (0, 0)),
        ],
        out_specs=[
            pl.BlockSpec((_BM1, H1), lambda i: (i, 0)),
            pl.BlockSpec((_BM1, H1), lambda i: (i, 0)),
        ],
        out_shape=[
            jax.ShapeDtypeStruct((N, H1), jnp.bfloat16),
            jax.ShapeDtypeStruct((N, H1), jnp.bfloat16),
        ],
    )(x, W0, W1)

    mw2 = pl.pallas_call(
        _layer1_body,
        grid=(pl.cdiv(N, _BM2),),
        in_specs=[
            pl.BlockSpec((1, _BM2, N), lambda i: (0, i, 0)),
            pl.BlockSpec((1, _BM2, N), lambda i: (1, i, 0)),
            pl.BlockSpec((N, H1), lambda i: (0, 0)),
            pl.BlockSpec((N, H1), lambda i: (0, 0)),
            pl.BlockSpec((H1, H2), lambda i: (0, 0)),
        ],
        out_specs=pl.BlockSpec((_BM2, H2), lambda i: (i, 0)),
        out_shape=jax.ShapeDtypeStruct((N, H2), jnp.bfloat16),
    )(adjs, adjs, p0, p1, W2)

    emb, embb, embt = pl.pallas_call(
        _layer2_body,
        grid=(pl.cdiv(N, _BM3),),
        in_specs=[
            pl.BlockSpec((1, _BM3, N), lambda i: (0, i, 0)),
            pl.BlockSpec((N, H2), lambda i: (0, 0)),
        ],
        out_specs=[
            pl.BlockSpec((_BM3, H2), lambda i: (i, 0)),
            pl.BlockSpec((_BM3, H2), lambda i: (i, 0)),
            pl.BlockSpec((H2, _BM3), lambda i: (0, i)),
        ],
        out_shape=[
            jax.ShapeDtypeStruct((N, H2), jnp.float32),
            jax.ShapeDtypeStruct((N, H2), jnp.bfloat16),
            jax.ShapeDtypeStruct((H2, N), jnp.bfloat16),
        ],
    )(adjs, mw2)

    recon0, recon1 = pl.pallas_call(
        _recon_body,
        grid=(pl.cdiv(N, _BTI), pl.cdiv(N, _BTJ)),
        in_specs=[
            pl.BlockSpec((_BTI, H2), lambda i, j: (i, 0)),
            pl.BlockSpec((H2, _BTJ), lambda i, j: (0, j)),
        ],
        out_specs=[
            pl.BlockSpec((_BTI, _BTJ), lambda i, j: (i, j)),
            pl.BlockSpec((_BTI, _BTJ), lambda i, j: (i, j)),
        ],
        out_shape=[
            jax.ShapeDtypeStruct((N, N), jnp.float32),
            jax.ShapeDtypeStruct((N, N), jnp.float32),
        ],
    )(embb, embt)

    return emb, recon0, recon1
